# baseline (speedup 1.0000x reference)
import jax
import jax.numpy as jnp
from jax import lax
from jax.experimental import pallas as pl
from jax.experimental.pallas import tpu as pltpu
from jax.experimental.pallas import tpu_sc as plsc

_BATCH = 16384
_F = 64
_NU = 1000000
_NC = 2
_NS = 16
_NW = _NC * _NS
_BPW = _BATCH // _NW
_CHUNK = 128
_NCHUNK = _BPW // _CHUNK
_NBLK = (_NU + 127) // 128
_LASTU0 = _NU - 128
_NPAIR = _NU // 2


def _wid():
    return lax.axis_index("s") * _NC + lax.axis_index("c")


def _detile_body(pt_hbm, qt_hbm, cp_hbm, cq_hbm,
                 insc, outsc, insc2, outsc2, s_in0, s_in1, s_out0, s_out1):
    wid = _wid()
    nfull = _NBLK - 1
    nblk = (nfull // _NW) + jnp.where(wid < nfull % _NW, 1, 0)

    lanes16 = lax.iota(jnp.int32, 16)
    uh = [(l0 * 16 + lanes16) >> 1 for l0 in range(8)]
    par6 = [((l0 * 16 + lanes16) & 1) << 6 for l0 in range(8)]

    for (src, dst, s_in, s_out) in ((pt_hbm, cp_hbm, s_in0, s_out0),
                                    (qt_hbm, cq_hbm, s_in1, s_out1)):
        def u0_of(k):
            return pl.multiple_of((wid + k * _NW) * 128, 128)

        def start_in(k, b):
            pltpu.make_async_copy(
                src.at[:, pl.ds(u0_of(k), 128)], insc.at[b], s_in).start()

        def wait_in(b):
            pltpu.make_async_copy(
                src.at[:, pl.ds(0, 128)], insc.at[b], s_in).wait()

        def start_out(k, b):
            r0 = pl.multiple_of(u0_of(k) >> 1, 8)
            pltpu.make_async_copy(
                outsc.at[b], dst.at[pl.ds(r0, 64), :], s_out).start()

        def wait_out(b):
            pltpu.make_async_copy(
                outsc.at[b], dst.at[pl.ds(0, 64), :], s_out).wait()

        start_in(0, 0)

        def blk(k, carry):
            b = k % 2

            @pl.when(k + 1 < nblk)
            def _():
                start_in(k + 1, (k + 1) % 2)

            wait_in(b)

            @pl.when(k >= 2)
            def _():
                wait_out(b)

            def frow(fr, c):
                for l0 in range(8):
                    v = insc[b, fr, pl.ds(l0 * 16, 16)]
                    plsc.store_scatter(outsc.at[b], [uh[l0], par6[l0] + fr], v)
                return c
            lax.fori_loop(0, 64, frow, 0)

            start_out(k, b)
            return carry

        lax.fori_loop(0, nblk, blk, 0)

        @pl.when(nblk >= 2)
        def _():
            wait_out(nblk % 2)
        wait_out((nblk + 1) % 2)

        @pl.when(wid == 0)
        def _():
            pltpu.sync_copy(src.at[:, pl.ds(_NBLK * 128 - 128, 64)], insc2)

            def frow2(fr, c):
                for l0 in range(4):
                    v = insc2[fr, pl.ds(l0 * 16, 16)]
                    plsc.store_scatter(outsc2, [uh[l0], par6[l0] + fr], v)
                return c
            lax.fori_loop(0, 64, frow2, 0)
            pltpu.sync_copy(outsc2, dst.at[pl.ds(_NPAIR - 32, 32), :])


def _bias_body(uid_hbm, iid_hbm, bu_hbm, bi_hbm, out_hbm,
               uidx, iidx, uhi, ihi, burows, birows, outv, sem):
    wid = _wid()
    base = wid * _BPW

    for j in range(_NCHUNK):
        pltpu.sync_copy(uid_hbm.at[pl.ds(base + j * _CHUNK, _CHUNK)], uidx.at[j])
        pltpu.sync_copy(iid_hbm.at[pl.ds(base + j * _CHUNK, _CHUNK)], iidx.at[j])

    for j in range(_NCHUNK):
        for t in range(_CHUNK // 16):
            sl = pl.ds(t * 16, 16)
            uhi.at[j][sl] = lax.shift_right_logical(uidx.at[j][sl], 4)
            ihi.at[j][sl] = lax.shift_right_logical(iidx.at[j][sl], 4)

    copies = []
    for j in range(_NCHUNK):
        sl = pl.ds(j * _CHUNK, _CHUNK)
        copies.append(pltpu.async_copy(bu_hbm.at[uhi.at[j]], burows.at[sl], sem))
        copies.append(pltpu.async_copy(bi_hbm.at[ihi.at[j]], birows.at[sl], sem))
    for cp in copies:
        cp.wait()

    lanes = lax.iota(jnp.int32, 16)

    def group(g, carry):
        rb = g * 16
        j = g // (_CHUNK // 16)
        o = (g % (_CHUNK // 16)) * 16
        rows = rb + lanes
        uvals = uidx.at[j][pl.ds(o, 16)]
        ivals = iidx.at[j][pl.ds(o, 16)]
        bu_v = plsc.load_gather(burows, [rows, jnp.bitwise_and(uvals, 15)])
        bi_v = plsc.load_gather(birows, [rows, jnp.bitwise_and(ivals, 15)])
        outv[pl.ds(rb, 16)] = bu_v + bi_v
        return carry

    lax.fori_loop(0, _BPW // 16, group, 0)
    pltpu.sync_copy(outv, out_hbm.at[pl.ds(base, _BPW)])


def _dot_body(uid_hbm, iid_hbm, cp_hbm, cq_hbm, bsum_hbm, out_hbm,
              uidx, iidx, upr, ipr, pbuf, qbuf, bsum, outv, s_p0, s_p1,
              s_q0, s_q1):
    wid = _wid()
    base = wid * _BPW

    for j in range(_NCHUNK):
        pltpu.sync_copy(uid_hbm.at[pl.ds(base + j * _CHUNK, _CHUNK)], uidx.at[j])
        pltpu.sync_copy(iid_hbm.at[pl.ds(base + j * _CHUNK, _CHUNK)], iidx.at[j])
    pltpu.sync_copy(bsum_hbm.at[pl.ds(base, _BPW)], bsum)

    for j in range(_NCHUNK):
        for t in range(_CHUNK // 16):
            sl = pl.ds(t * 16, 16)
            upr.at[j][sl] = lax.shift_right_logical(uidx.at[j][sl], 1)
            ipr.at[j][sl] = lax.shift_right_logical(iidx.at[j][sl], 1)

    sems = ((s_p0, s_q0), (s_p1, s_q1))

    def fire(j):
        b = j % 2
        pltpu.make_async_copy(cp_hbm.at[upr.at[j]], pbuf.at[b], sems[b][0]).start()
        pltpu.make_async_copy(cq_hbm.at[ipr.at[j]], qbuf.at[b], sems[b][1]).start()

    def drain(b):
        pltpu.make_async_copy(cp_hbm.at[upr.at[0]], pbuf.at[b], sems[b][0]).wait()
        pltpu.make_async_copy(cq_hbm.at[ipr.at[0]], qbuf.at[b], sems[b][1]).wait()

    lanes = lax.iota(jnp.int32, 16)
    fire(0)
    for j in range(_NCHUNK):
        if j + 1 < _NCHUNK:
            fire(j + 1)
        b = j % 2
        drain(b)
        for g in range(_CHUNK // 16):
            uvals = uidx.at[j][pl.ds(g * 16, 16)]
            ivals = iidx.at[j][pl.ds(g * 16, 16)]
            sums = bsum[pl.ds(j * _CHUNK + g * 16, 16)]
            for i in range(16):
                r = g * 16 + i
                hu = jnp.bitwise_and(uvals[i], 1) * 64
                hi_ = jnp.bitwise_and(ivals[i], 1) * 64
                a = (pbuf[b, r, pl.ds(hu, 16)] * qbuf[b, r, pl.ds(hi_, 16)])
                for k in range(1, _F // 16):
                    a = a + (pbuf[b, r, pl.ds(hu + 16 * k, 16)]
                             * qbuf[b, r, pl.ds(hi_ + 16 * k, 16)])
                sums = jnp.where(lanes == i, jnp.sum(a) + sums, sums)
            outv[pl.ds(j * _CHUNK + g * 16, 16)] = sums

    pltpu.sync_copy(outv, out_hbm.at[pl.ds(base, _BPW)])


@jax.jit
def kernel(user_id, item_id, P, Q, user_bias, item_bias):
    mesh = plsc.VectorSubcoreMesh(core_axis_name="c", subcore_axis_name="s")

    detile = pl.kernel(
        _detile_body,
        out_type=(jax.ShapeDtypeStruct((_NPAIR, 128), jnp.float32),
                  jax.ShapeDtypeStruct((_NPAIR, 128), jnp.float32)),
        mesh=mesh,
        compiler_params=pltpu.CompilerParams(
            needs_layout_passes=False, use_tc_tiling_on_sc=True),
        scratch_types=[
            pltpu.VMEM((2, 64, 128), jnp.float32),
            pltpu.VMEM((2, 64, 128), jnp.float32),
            pltpu.VMEM((64, 64), jnp.float32),
            pltpu.VMEM((32, 128), jnp.float32),
            pltpu.SemaphoreType.DMA,
            pltpu.SemaphoreType.DMA,
            pltpu.SemaphoreType.DMA,
            pltpu.SemaphoreType.DMA,
        ],
    )
    cp, cq = detile(P.T, Q.T)

    bias = pl.kernel(
        _bias_body,
        out_type=jax.ShapeDtypeStruct((_BATCH,), jnp.float32),
        mesh=mesh,
        compiler_params=pltpu.CompilerParams(
            needs_layout_passes=False, use_tc_tiling_on_sc=False),
        scratch_types=[
            pltpu.VMEM((_NCHUNK, _CHUNK), jnp.int32),
            pltpu.VMEM((_NCHUNK, _CHUNK), jnp.int32),
            pltpu.VMEM((_NCHUNK, _CHUNK), jnp.int32),
            pltpu.VMEM((_NCHUNK, _CHUNK), jnp.int32),
            pltpu.VMEM((_BPW, 16), jnp.float32),
            pltpu.VMEM((_BPW, 16), jnp.float32),
            pltpu.VMEM((_BPW,), jnp.float32),
            pltpu.SemaphoreType.DMA,
        ],
    )
    bsum = bias(user_id, item_id,
                user_bias.reshape(-1, 16), item_bias.reshape(-1, 16))

    dots = pl.kernel(
        _dot_body,
        out_type=jax.ShapeDtypeStruct((_BATCH,), jnp.float32),
        mesh=mesh,
        compiler_params=pltpu.CompilerParams(
            needs_layout_passes=False, use_tc_tiling_on_sc=True),
        scratch_types=[
            pltpu.VMEM((_NCHUNK, _CHUNK), jnp.int32),
            pltpu.VMEM((_NCHUNK, _CHUNK), jnp.int32),
            pltpu.VMEM((_NCHUNK, _CHUNK), jnp.int32),
            pltpu.VMEM((_NCHUNK, _CHUNK), jnp.int32),
            pltpu.VMEM((2, _CHUNK, 128), jnp.float32),
            pltpu.VMEM((2, _CHUNK, 128), jnp.float32),
            pltpu.VMEM((_BPW,), jnp.float32),
            pltpu.VMEM((_BPW,), jnp.float32),
            pltpu.SemaphoreType.DMA,
            pltpu.SemaphoreType.DMA,
            pltpu.SemaphoreType.DMA,
            pltpu.SemaphoreType.DMA,
        ],
    )
    return dots(user_id, item_id, cp, cq, bsum)


# speedup vs baseline: 1.3849x; 1.3849x over previous
import jax
import jax.numpy as jnp
from jax import lax
from jax.experimental import pallas as pl
from jax.experimental.pallas import tpu as pltpu
from jax.experimental.pallas import tpu_sc as plsc

_BATCH = 16384
_F = 64
_NU = 1000000
_NC = 2
_NS = 16
_NW = _NC * _NS
_BPW = _BATCH // _NW
_CHUNK = 128
_NCHUNK = _BPW // _CHUNK
_NBLK = (_NU + 127) // 128
_LASTU0 = _NU - 128
_NPAIR = _NU // 2


def _wid():
    return lax.axis_index("s") * _NC + lax.axis_index("c")


def _detile_body(pt_hbm, qt_hbm, cp_hbm, cq_hbm,
                 insc, outsc, insc2, outsc2, s_in0, s_in1, s_out0, s_out1):
    wid = _wid()
    nfull = _NBLK - 1
    nblk = (nfull // _NW) + jnp.where(wid < nfull % _NW, 1, 0)

    lanes16 = lax.iota(jnp.int32, 16)
    uh = [(l0 * 16 + lanes16) >> 1 for l0 in range(8)]
    par6 = [((l0 * 16 + lanes16) & 1) << 6 for l0 in range(8)]

    for (src, dst, s_in, s_out) in ((pt_hbm, cp_hbm, s_in0, s_out0),
                                    (qt_hbm, cq_hbm, s_in1, s_out1)):
        def u0_of(k):
            return pl.multiple_of((wid + k * _NW) * 128, 128)

        def start_in(k, b):
            pltpu.make_async_copy(
                src.at[:, pl.ds(u0_of(k), 128)], insc.at[b], s_in).start()

        def wait_in(b):
            pltpu.make_async_copy(
                src.at[:, pl.ds(0, 128)], insc.at[b], s_in).wait()

        def start_out(k, b):
            r0 = pl.multiple_of(u0_of(k) >> 1, 8)
            pltpu.make_async_copy(
                outsc.at[b], dst.at[pl.ds(r0, 64), :], s_out).start()

        def wait_out(b):
            pltpu.make_async_copy(
                outsc.at[b], dst.at[pl.ds(0, 64), :], s_out).wait()

        start_in(0, 0)

        def blk(k, carry):
            b = k % 2

            @pl.when(k + 1 < nblk)
            def _():
                start_in(k + 1, (k + 1) % 2)

            wait_in(b)

            @pl.when(k >= 2)
            def _():
                wait_out(b)

            @plsc.parallel_loop(0, 64, step=1, unroll=8)
            def frow(fr):
                for l0 in range(8):
                    v = insc[b, fr, pl.ds(l0 * 16, 16)]
                    plsc.store_scatter(outsc.at[b], [uh[l0], par6[l0] + fr], v)

            start_out(k, b)
            return carry

        lax.fori_loop(0, nblk, blk, 0)

        @pl.when(nblk >= 2)
        def _():
            wait_out(nblk % 2)
        wait_out((nblk + 1) % 2)

        @pl.when(wid == 0)
        def _():
            pltpu.sync_copy(src.at[:, pl.ds(_NBLK * 128 - 128, 64)], insc2)

            @plsc.parallel_loop(0, 64, step=1, unroll=8)
            def frow2(fr):
                for l0 in range(4):
                    v = insc2[fr, pl.ds(l0 * 16, 16)]
                    plsc.store_scatter(outsc2, [uh[l0], par6[l0] + fr], v)
            pltpu.sync_copy(outsc2, dst.at[pl.ds(_NPAIR - 32, 32), :])


def _bias_body(uid_hbm, iid_hbm, bu_hbm, bi_hbm, out_hbm,
               uidx, iidx, uhi, ihi, burows, birows, outv, sem):
    wid = _wid()
    base = wid * _BPW

    for j in range(_NCHUNK):
        pltpu.sync_copy(uid_hbm.at[pl.ds(base + j * _CHUNK, _CHUNK)], uidx.at[j])
        pltpu.sync_copy(iid_hbm.at[pl.ds(base + j * _CHUNK, _CHUNK)], iidx.at[j])

    for j in range(_NCHUNK):
        for t in range(_CHUNK // 16):
            sl = pl.ds(t * 16, 16)
            uhi.at[j][sl] = lax.shift_right_logical(uidx.at[j][sl], 4)
            ihi.at[j][sl] = lax.shift_right_logical(iidx.at[j][sl], 4)

    copies = []
    for j in range(_NCHUNK):
        sl = pl.ds(j * _CHUNK, _CHUNK)
        copies.append(pltpu.async_copy(bu_hbm.at[uhi.at[j]], burows.at[sl], sem))
        copies.append(pltpu.async_copy(bi_hbm.at[ihi.at[j]], birows.at[sl], sem))
    for cp in copies:
        cp.wait()

    lanes = lax.iota(jnp.int32, 16)

    def group(g, carry):
        rb = g * 16
        j = g // (_CHUNK // 16)
        o = (g % (_CHUNK // 16)) * 16
        rows = rb + lanes
        uvals = uidx.at[j][pl.ds(o, 16)]
        ivals = iidx.at[j][pl.ds(o, 16)]
        bu_v = plsc.load_gather(burows, [rows, jnp.bitwise_and(uvals, 15)])
        bi_v = plsc.load_gather(birows, [rows, jnp.bitwise_and(ivals, 15)])
        outv[pl.ds(rb, 16)] = bu_v + bi_v
        return carry

    lax.fori_loop(0, _BPW // 16, group, 0)
    pltpu.sync_copy(outv, out_hbm.at[pl.ds(base, _BPW)])


def _dot_body(uid_hbm, iid_hbm, cp_hbm, cq_hbm, bsum_hbm, out_hbm,
              uidx, iidx, upr, ipr, pbuf, qbuf, bsum, outv, s_p0, s_p1,
              s_q0, s_q1):
    wid = _wid()
    base = wid * _BPW

    for j in range(_NCHUNK):
        pltpu.sync_copy(uid_hbm.at[pl.ds(base + j * _CHUNK, _CHUNK)], uidx.at[j])
        pltpu.sync_copy(iid_hbm.at[pl.ds(base + j * _CHUNK, _CHUNK)], iidx.at[j])
    pltpu.sync_copy(bsum_hbm.at[pl.ds(base, _BPW)], bsum)

    for j in range(_NCHUNK):
        for t in range(_CHUNK // 16):
            sl = pl.ds(t * 16, 16)
            upr.at[j][sl] = lax.shift_right_logical(uidx.at[j][sl], 1)
            ipr.at[j][sl] = lax.shift_right_logical(iidx.at[j][sl], 1)

    sems = ((s_p0, s_q0), (s_p1, s_q1))

    def fire(j):
        b = j % 2
        pltpu.make_async_copy(cp_hbm.at[upr.at[j]], pbuf.at[b], sems[b][0]).start()
        pltpu.make_async_copy(cq_hbm.at[ipr.at[j]], qbuf.at[b], sems[b][1]).start()

    def drain(b):
        pltpu.make_async_copy(cp_hbm.at[upr.at[0]], pbuf.at[b], sems[b][0]).wait()
        pltpu.make_async_copy(cq_hbm.at[ipr.at[0]], qbuf.at[b], sems[b][1]).wait()

    lanes = lax.iota(jnp.int32, 16)
    fire(0)
    for j in range(_NCHUNK):
        if j + 1 < _NCHUNK:
            fire(j + 1)
        b = j % 2
        drain(b)
        for g in range(_CHUNK // 16):
            uvals = uidx.at[j][pl.ds(g * 16, 16)]
            ivals = iidx.at[j][pl.ds(g * 16, 16)]
            sums = bsum[pl.ds(j * _CHUNK + g * 16, 16)]
            for i in range(16):
                r = g * 16 + i
                hu = jnp.bitwise_and(uvals[i], 1) * 64
                hi_ = jnp.bitwise_and(ivals[i], 1) * 64
                a = (pbuf[b, r, pl.ds(hu, 16)] * qbuf[b, r, pl.ds(hi_, 16)])
                for k in range(1, _F // 16):
                    a = a + (pbuf[b, r, pl.ds(hu + 16 * k, 16)]
                             * qbuf[b, r, pl.ds(hi_ + 16 * k, 16)])
                sums = jnp.where(lanes == i, jnp.sum(a) + sums, sums)
            outv[pl.ds(j * _CHUNK + g * 16, 16)] = sums

    pltpu.sync_copy(outv, out_hbm.at[pl.ds(base, _BPW)])


@jax.jit
def kernel(user_id, item_id, P, Q, user_bias, item_bias):
    mesh = plsc.VectorSubcoreMesh(core_axis_name="c", subcore_axis_name="s")

    detile = pl.kernel(
        _detile_body,
        out_type=(jax.ShapeDtypeStruct((_NPAIR, 128), jnp.float32),
                  jax.ShapeDtypeStruct((_NPAIR, 128), jnp.float32)),
        mesh=mesh,
        compiler_params=pltpu.CompilerParams(
            needs_layout_passes=False, use_tc_tiling_on_sc=True),
        scratch_types=[
            pltpu.VMEM((2, 64, 128), jnp.float32),
            pltpu.VMEM((2, 64, 128), jnp.float32),
            pltpu.VMEM((64, 64), jnp.float32),
            pltpu.VMEM((32, 128), jnp.float32),
            pltpu.SemaphoreType.DMA,
            pltpu.SemaphoreType.DMA,
            pltpu.SemaphoreType.DMA,
            pltpu.SemaphoreType.DMA,
        ],
    )
    cp, cq = detile(P.T, Q.T)

    bias = pl.kernel(
        _bias_body,
        out_type=jax.ShapeDtypeStruct((_BATCH,), jnp.float32),
        mesh=mesh,
        compiler_params=pltpu.CompilerParams(
            needs_layout_passes=False, use_tc_tiling_on_sc=False),
        scratch_types=[
            pltpu.VMEM((_NCHUNK, _CHUNK), jnp.int32),
            pltpu.VMEM((_NCHUNK, _CHUNK), jnp.int32),
            pltpu.VMEM((_NCHUNK, _CHUNK), jnp.int32),
            pltpu.VMEM((_NCHUNK, _CHUNK), jnp.int32),
            pltpu.VMEM((_BPW, 16), jnp.float32),
            pltpu.VMEM((_BPW, 16), jnp.float32),
            pltpu.VMEM((_BPW,), jnp.float32),
            pltpu.SemaphoreType.DMA,
        ],
    )
    bsum = bias(user_id, item_id,
                user_bias.reshape(-1, 16), item_bias.reshape(-1, 16))

    dots = pl.kernel(
        _dot_body,
        out_type=jax.ShapeDtypeStruct((_BATCH,), jnp.float32),
        mesh=mesh,
        compiler_params=pltpu.CompilerParams(
            needs_layout_passes=False, use_tc_tiling_on_sc=True),
        scratch_types=[
            pltpu.VMEM((_NCHUNK, _CHUNK), jnp.int32),
            pltpu.VMEM((_NCHUNK, _CHUNK), jnp.int32),
            pltpu.VMEM((_NCHUNK, _CHUNK), jnp.int32),
            pltpu.VMEM((_NCHUNK, _CHUNK), jnp.int32),
            pltpu.VMEM((2, _CHUNK, 128), jnp.float32),
            pltpu.VMEM((2, _CHUNK, 128), jnp.float32),
            pltpu.VMEM((_BPW,), jnp.float32),
            pltpu.VMEM((_BPW,), jnp.float32),
            pltpu.SemaphoreType.DMA,
            pltpu.SemaphoreType.DMA,
            pltpu.SemaphoreType.DMA,
            pltpu.SemaphoreType.DMA,
        ],
    )
    return dots(user_id, item_id, cp, cq, bsum)


# speedup vs baseline: 4.2314x; 3.0554x over previous
import jax
import jax.numpy as jnp
from jax import lax
from jax.experimental import pallas as pl
from jax.experimental.pallas import tpu as pltpu
from jax.experimental.pallas import tpu_sc as plsc

_BATCH = 16384
_F = 64
_NU = 1000000
_NC = 2
_NS = 16
_NW = _NC * _NS
_BPW = _BATCH // _NW
_CHUNK = 128
_NCHUNK = _BPW // _CHUNK
_NBLK = (_NU + 127) // 128
_LASTU0 = _NU - 128
_NPAIR = _NU // 2


def _wid():
    return lax.axis_index("s") * _NC + lax.axis_index("c")


def _detile_body(pt_hbm, qt_hbm, cp_hbm, cq_hbm,
                 insc, outsc, insc2, outsc2, s_in0, s_in1, s_out0, s_out1):
    wid = _wid()
    nfull = _NBLK - 1
    nblk = (nfull // _NW) + jnp.where(wid < nfull % _NW, 1, 0)

    lanes16 = lax.iota(jnp.int32, 16)
    uh = [(l0 * 16 + lanes16) >> 1 for l0 in range(8)]
    par6 = [((l0 * 16 + lanes16) & 1) << 6 for l0 in range(8)]
    fdiag = [(lanes16 + d) & 15 for d in range(16)]
    upar = (lanes16 & 1) << 6
    uhalf = lanes16 >> 1

    for (src, dst, s_in, s_out) in ((pt_hbm, cp_hbm, s_in0, s_out0),
                                    (qt_hbm, cq_hbm, s_in1, s_out1)):
        def u0_of(k):
            return pl.multiple_of((wid + k * _NW) * 128, 128)

        def start_in(k, b):
            pltpu.make_async_copy(
                src.at[:, pl.ds(u0_of(k), 128)], insc.at[b], s_in).start()

        def wait_in(b):
            pltpu.make_async_copy(
                src.at[:, pl.ds(0, 128)], insc.at[b], s_in).wait()

        def start_out(k, b):
            r0 = pl.multiple_of(u0_of(k) >> 1, 8)
            pltpu.make_async_copy(
                outsc.at[b], dst.at[pl.ds(r0, 64), :], s_out).start()

        def wait_out(b):
            pltpu.make_async_copy(
                outsc.at[b], dst.at[pl.ds(0, 64), :], s_out).wait()

        start_in(0, 0)

        def blk(k, carry):
            b = k % 2

            @pl.when(k + 1 < nblk)
            def _():
                start_in(k + 1, (k + 1) % 2)

            wait_in(b)

            @pl.when(k >= 2)
            def _():
                wait_out(b)

            @plsc.parallel_loop(0, 16, step=1, unroll=4)
            def frow(d):
                fd = (lanes16 + d) & 15
                for u0 in range(0, 128, 16):
                    u_vec = u0 + lanes16
                    r_vec = (u0 >> 1) + uhalf
                    for f0 in range(0, 64, 16):
                        f_vec = f0 + fd
                        v = plsc.load_gather(insc.at[b], [f_vec, u_vec])
                        plsc.store_scatter(outsc.at[b], [r_vec, f_vec + upar], v)

            start_out(k, b)
            return carry

        lax.fori_loop(0, nblk, blk, 0)

        @pl.when(nblk >= 2)
        def _():
            wait_out(nblk % 2)
        wait_out((nblk + 1) % 2)

        @pl.when(wid == 0)
        def _():
            pltpu.sync_copy(src.at[:, pl.ds(_NBLK * 128 - 128, 64)], insc2)

            @plsc.parallel_loop(0, 64, step=1, unroll=8)
            def frow2(fr):
                for l0 in range(4):
                    v = insc2[fr, pl.ds(l0 * 16, 16)]
                    plsc.store_scatter(outsc2, [uh[l0], par6[l0] + fr], v)
            pltpu.sync_copy(outsc2, dst.at[pl.ds(_NPAIR - 32, 32), :])


def _bias_body(uid_hbm, iid_hbm, bu_hbm, bi_hbm, out_hbm,
               uidx, iidx, uhi, ihi, burows, birows, outv, sem):
    wid = _wid()
    base = wid * _BPW

    for j in range(_NCHUNK):
        pltpu.sync_copy(uid_hbm.at[pl.ds(base + j * _CHUNK, _CHUNK)], uidx.at[j])
        pltpu.sync_copy(iid_hbm.at[pl.ds(base + j * _CHUNK, _CHUNK)], iidx.at[j])

    for j in range(_NCHUNK):
        for t in range(_CHUNK // 16):
            sl = pl.ds(t * 16, 16)
            uhi.at[j][sl] = lax.shift_right_logical(uidx.at[j][sl], 4)
            ihi.at[j][sl] = lax.shift_right_logical(iidx.at[j][sl], 4)

    copies = []
    for j in range(_NCHUNK):
        sl = pl.ds(j * _CHUNK, _CHUNK)
        copies.append(pltpu.async_copy(bu_hbm.at[uhi.at[j]], burows.at[sl], sem))
        copies.append(pltpu.async_copy(bi_hbm.at[ihi.at[j]], birows.at[sl], sem))
    for cp in copies:
        cp.wait()

    lanes = lax.iota(jnp.int32, 16)

    def group(g, carry):
        rb = g * 16
        j = g // (_CHUNK // 16)
        o = (g % (_CHUNK // 16)) * 16
        rows = rb + lanes
        uvals = uidx.at[j][pl.ds(o, 16)]
        ivals = iidx.at[j][pl.ds(o, 16)]
        bu_v = plsc.load_gather(burows, [rows, jnp.bitwise_and(uvals, 15)])
        bi_v = plsc.load_gather(birows, [rows, jnp.bitwise_and(ivals, 15)])
        outv[pl.ds(rb, 16)] = bu_v + bi_v
        return carry

    lax.fori_loop(0, _BPW // 16, group, 0)
    pltpu.sync_copy(outv, out_hbm.at[pl.ds(base, _BPW)])


def _dot_body(uid_hbm, iid_hbm, cp_hbm, cq_hbm, bsum_hbm, out_hbm,
              uidx, iidx, upr, ipr, pbuf, qbuf, bsum, outv, s_p0, s_p1,
              s_q0, s_q1):
    wid = _wid()
    base = wid * _BPW

    for j in range(_NCHUNK):
        pltpu.sync_copy(uid_hbm.at[pl.ds(base + j * _CHUNK, _CHUNK)], uidx.at[j])
        pltpu.sync_copy(iid_hbm.at[pl.ds(base + j * _CHUNK, _CHUNK)], iidx.at[j])
    pltpu.sync_copy(bsum_hbm.at[pl.ds(base, _BPW)], bsum)

    for j in range(_NCHUNK):
        for t in range(_CHUNK // 16):
            sl = pl.ds(t * 16, 16)
            upr.at[j][sl] = lax.shift_right_logical(uidx.at[j][sl], 1)
            ipr.at[j][sl] = lax.shift_right_logical(iidx.at[j][sl], 1)

    sems = ((s_p0, s_q0), (s_p1, s_q1))

    def fire(j):
        b = j % 2
        pltpu.make_async_copy(cp_hbm.at[upr.at[j]], pbuf.at[b], sems[b][0]).start()
        pltpu.make_async_copy(cq_hbm.at[ipr.at[j]], qbuf.at[b], sems[b][1]).start()

    def drain(b):
        pltpu.make_async_copy(cp_hbm.at[upr.at[0]], pbuf.at[b], sems[b][0]).wait()
        pltpu.make_async_copy(cq_hbm.at[ipr.at[0]], qbuf.at[b], sems[b][1]).wait()

    lanes = lax.iota(jnp.int32, 16)
    fire(0)
    for j in range(_NCHUNK):
        if j + 1 < _NCHUNK:
            fire(j + 1)
        b = j % 2
        drain(b)
        for g in range(_CHUNK // 16):
            uvals = uidx.at[j][pl.ds(g * 16, 16)]
            ivals = iidx.at[j][pl.ds(g * 16, 16)]
            sums = bsum[pl.ds(j * _CHUNK + g * 16, 16)]
            for i in range(16):
                r = g * 16 + i
                hu = jnp.bitwise_and(uvals[i], 1) * 64
                hi_ = jnp.bitwise_and(ivals[i], 1) * 64
                a = (pbuf[b, r, pl.ds(hu, 16)] * qbuf[b, r, pl.ds(hi_, 16)])
                for k in range(1, _F // 16):
                    a = a + (pbuf[b, r, pl.ds(hu + 16 * k, 16)]
                             * qbuf[b, r, pl.ds(hi_ + 16 * k, 16)])
                sums = jnp.where(lanes == i, jnp.sum(a) + sums, sums)
            outv[pl.ds(j * _CHUNK + g * 16, 16)] = sums

    pltpu.sync_copy(outv, out_hbm.at[pl.ds(base, _BPW)])


@jax.jit
def kernel(user_id, item_id, P, Q, user_bias, item_bias):
    mesh = plsc.VectorSubcoreMesh(core_axis_name="c", subcore_axis_name="s")

    detile = pl.kernel(
        _detile_body,
        out_type=(jax.ShapeDtypeStruct((_NPAIR, 128), jnp.float32),
                  jax.ShapeDtypeStruct((_NPAIR, 128), jnp.float32)),
        mesh=mesh,
        compiler_params=pltpu.CompilerParams(
            needs_layout_passes=False, use_tc_tiling_on_sc=True),
        scratch_types=[
            pltpu.VMEM((2, 64, 128), jnp.float32),
            pltpu.VMEM((2, 64, 128), jnp.float32),
            pltpu.VMEM((64, 64), jnp.float32),
            pltpu.VMEM((32, 128), jnp.float32),
            pltpu.SemaphoreType.DMA,
            pltpu.SemaphoreType.DMA,
            pltpu.SemaphoreType.DMA,
            pltpu.SemaphoreType.DMA,
        ],
    )
    cp, cq = detile(P.T, Q.T)

    bias = pl.kernel(
        _bias_body,
        out_type=jax.ShapeDtypeStruct((_BATCH,), jnp.float32),
        mesh=mesh,
        compiler_params=pltpu.CompilerParams(
            needs_layout_passes=False, use_tc_tiling_on_sc=False),
        scratch_types=[
            pltpu.VMEM((_NCHUNK, _CHUNK), jnp.int32),
            pltpu.VMEM((_NCHUNK, _CHUNK), jnp.int32),
            pltpu.VMEM((_NCHUNK, _CHUNK), jnp.int32),
            pltpu.VMEM((_NCHUNK, _CHUNK), jnp.int32),
            pltpu.VMEM((_BPW, 16), jnp.float32),
            pltpu.VMEM((_BPW, 16), jnp.float32),
            pltpu.VMEM((_BPW,), jnp.float32),
            pltpu.SemaphoreType.DMA,
        ],
    )
    bsum = bias(user_id, item_id,
                user_bias.reshape(-1, 16), item_bias.reshape(-1, 16))

    dots = pl.kernel(
        _dot_body,
        out_type=jax.ShapeDtypeStruct((_BATCH,), jnp.float32),
        mesh=mesh,
        compiler_params=pltpu.CompilerParams(
            needs_layout_passes=False, use_tc_tiling_on_sc=True),
        scratch_types=[
            pltpu.VMEM((_NCHUNK, _CHUNK), jnp.int32),
            pltpu.VMEM((_NCHUNK, _CHUNK), jnp.int32),
            pltpu.VMEM((_NCHUNK, _CHUNK), jnp.int32),
            pltpu.VMEM((_NCHUNK, _CHUNK), jnp.int32),
            pltpu.VMEM((2, _CHUNK, 128), jnp.float32),
            pltpu.VMEM((2, _CHUNK, 128), jnp.float32),
            pltpu.VMEM((_BPW,), jnp.float32),
            pltpu.VMEM((_BPW,), jnp.float32),
            pltpu.SemaphoreType.DMA,
            pltpu.SemaphoreType.DMA,
            pltpu.SemaphoreType.DMA,
            pltpu.SemaphoreType.DMA,
        ],
    )
    return dots(user_id, item_id, cp, cq, bsum)


# speedup vs baseline: 4.7231x; 1.1162x over previous
import jax
import jax.numpy as jnp
from jax import lax
from jax.experimental import pallas as pl
from jax.experimental.pallas import tpu as pltpu
from jax.experimental.pallas import tpu_sc as plsc

_BATCH = 16384
_F = 64
_NU = 1000000
_NC = 2
_NS = 16
_NW = _NC * _NS
_BPW = _BATCH // _NW
_CHUNK = 128
_NCHUNK = _BPW // _CHUNK
_NBLK = (_NU + 127) // 128
_LASTU0 = _NU - 128
_NPAIR = _NU // 2


def _wid():
    return lax.axis_index("s") * _NC + lax.axis_index("c")


def _detile_body(pt_hbm, qt_hbm, cp_hbm, cq_hbm,
                 insc, outsc, insc2, outsc2, s_in0, s_in1, s_out0, s_out1):
    wid = _wid()
    nfull = _NU // 256
    nblk = (nfull // _NW) + jnp.where(wid < nfull % _NW, 1, 0)

    lanes16 = lax.iota(jnp.int32, 16)
    uh = [(l0 * 16 + lanes16) >> 1 for l0 in range(8)]
    par6 = [((l0 * 16 + lanes16) & 1) << 6 for l0 in range(8)]
    fdiag = [(lanes16 + d) & 15 for d in range(16)]
    upar = (lanes16 & 1) << 6
    uhalf = lanes16 >> 1

    for (src, dst, s_in, s_out) in ((pt_hbm, cp_hbm, s_in0, s_out0),
                                    (qt_hbm, cq_hbm, s_in1, s_out1)):
        def u0_of(k):
            return pl.multiple_of((wid + k * _NW) * 256, 128)

        def start_in(k, b):
            pltpu.make_async_copy(
                src.at[:, pl.ds(u0_of(k), 256)], insc.at[b], s_in).start()

        def wait_in(b):
            pltpu.make_async_copy(
                src.at[:, pl.ds(0, 128)], insc.at[b], s_in).wait()

        def start_out(k, b):
            r0 = pl.multiple_of(u0_of(k) >> 1, 8)
            pltpu.make_async_copy(
                outsc.at[b], dst.at[pl.ds(r0, 128), :], s_out).start()

        def wait_out(b):
            pltpu.make_async_copy(
                outsc.at[b], dst.at[pl.ds(0, 128), :], s_out).wait()

        start_in(0, 0)

        def blk(k, carry):
            b = k % 2

            @pl.when(k + 1 < nblk)
            def _():
                start_in(k + 1, (k + 1) % 2)

            wait_in(b)

            @pl.when(k >= 2)
            def _():
                wait_out(b)

            @plsc.parallel_loop(0, 16, step=1, unroll=8)
            def frow(d):
                fd = (lanes16 + d) & 15
                for u0 in range(0, 256, 16):
                    u_vec = u0 + lanes16
                    r_vec = (u0 >> 1) + uhalf
                    for f0 in range(0, 64, 16):
                        f_vec = f0 + fd
                        v = plsc.load_gather(insc.at[b], [f_vec, u_vec])
                        plsc.store_scatter(outsc.at[b], [r_vec, f_vec + upar], v)

            start_out(k, b)
            return carry

        lax.fori_loop(0, nblk, blk, 0)

        @pl.when(nblk >= 2)
        def _():
            wait_out(nblk % 2)
        wait_out((nblk + 1) % 2)

        @pl.when(wid == 0)
        def _():
            pltpu.sync_copy(src.at[:, pl.ds(_NU - 64, 64)], insc2)

            @plsc.parallel_loop(0, 64, step=1, unroll=8)
            def frow2(fr):
                for l0 in range(4):
                    v = insc2[fr, pl.ds(l0 * 16, 16)]
                    plsc.store_scatter(outsc2, [uh[l0], par6[l0] + fr], v)
            pltpu.sync_copy(outsc2, dst.at[pl.ds(_NPAIR - 32, 32), :])


def _bias_body(uid_hbm, iid_hbm, bu_hbm, bi_hbm, out_hbm,
               uidx, iidx, uhi, ihi, burows, birows, outv, sem):
    wid = _wid()
    base = wid * _BPW

    for j in range(_NCHUNK):
        pltpu.sync_copy(uid_hbm.at[pl.ds(base + j * _CHUNK, _CHUNK)], uidx.at[j])
        pltpu.sync_copy(iid_hbm.at[pl.ds(base + j * _CHUNK, _CHUNK)], iidx.at[j])

    for j in range(_NCHUNK):
        for t in range(_CHUNK // 16):
            sl = pl.ds(t * 16, 16)
            uhi.at[j][sl] = lax.shift_right_logical(uidx.at[j][sl], 4)
            ihi.at[j][sl] = lax.shift_right_logical(iidx.at[j][sl], 4)

    copies = []
    for j in range(_NCHUNK):
        sl = pl.ds(j * _CHUNK, _CHUNK)
        copies.append(pltpu.async_copy(bu_hbm.at[uhi.at[j]], burows.at[sl], sem))
        copies.append(pltpu.async_copy(bi_hbm.at[ihi.at[j]], birows.at[sl], sem))
    for cp in copies:
        cp.wait()

    lanes = lax.iota(jnp.int32, 16)

    def group(g, carry):
        rb = g * 16
        j = g // (_CHUNK // 16)
        o = (g % (_CHUNK // 16)) * 16
        rows = rb + lanes
        uvals = uidx.at[j][pl.ds(o, 16)]
        ivals = iidx.at[j][pl.ds(o, 16)]
        bu_v = plsc.load_gather(burows, [rows, jnp.bitwise_and(uvals, 15)])
        bi_v = plsc.load_gather(birows, [rows, jnp.bitwise_and(ivals, 15)])
        outv[pl.ds(rb, 16)] = bu_v + bi_v
        return carry

    lax.fori_loop(0, _BPW // 16, group, 0)
    pltpu.sync_copy(outv, out_hbm.at[pl.ds(base, _BPW)])


def _dot_body(uid_hbm, iid_hbm, cp_hbm, cq_hbm, bsum_hbm, out_hbm,
              uidx, iidx, upr, ipr, pbuf, qbuf, bsum, outv, s_p0, s_p1,
              s_q0, s_q1):
    wid = _wid()
    base = wid * _BPW

    for j in range(_NCHUNK):
        pltpu.sync_copy(uid_hbm.at[pl.ds(base + j * _CHUNK, _CHUNK)], uidx.at[j])
        pltpu.sync_copy(iid_hbm.at[pl.ds(base + j * _CHUNK, _CHUNK)], iidx.at[j])
    pltpu.sync_copy(bsum_hbm.at[pl.ds(base, _BPW)], bsum)

    for j in range(_NCHUNK):
        for t in range(_CHUNK // 16):
            sl = pl.ds(t * 16, 16)
            upr.at[j][sl] = lax.shift_right_logical(uidx.at[j][sl], 1)
            ipr.at[j][sl] = lax.shift_right_logical(iidx.at[j][sl], 1)

    sems = ((s_p0, s_q0), (s_p1, s_q1))

    def fire(j):
        b = j % 2
        pltpu.make_async_copy(cp_hbm.at[upr.at[j]], pbuf.at[b], sems[b][0]).start()
        pltpu.make_async_copy(cq_hbm.at[ipr.at[j]], qbuf.at[b], sems[b][1]).start()

    def drain(b):
        pltpu.make_async_copy(cp_hbm.at[upr.at[0]], pbuf.at[b], sems[b][0]).wait()
        pltpu.make_async_copy(cq_hbm.at[ipr.at[0]], qbuf.at[b], sems[b][1]).wait()

    lanes = lax.iota(jnp.int32, 16)
    fire(0)
    for j in range(_NCHUNK):
        if j + 1 < _NCHUNK:
            fire(j + 1)
        b = j % 2
        drain(b)
        for g in range(_CHUNK // 16):
            uvals = uidx.at[j][pl.ds(g * 16, 16)]
            ivals = iidx.at[j][pl.ds(g * 16, 16)]
            sums = bsum[pl.ds(j * _CHUNK + g * 16, 16)]
            for i in range(16):
                r = g * 16 + i
                hu = jnp.bitwise_and(uvals[i], 1) * 64
                hi_ = jnp.bitwise_and(ivals[i], 1) * 64
                a = (pbuf[b, r, pl.ds(hu, 16)] * qbuf[b, r, pl.ds(hi_, 16)])
                for k in range(1, _F // 16):
                    a = a + (pbuf[b, r, pl.ds(hu + 16 * k, 16)]
                             * qbuf[b, r, pl.ds(hi_ + 16 * k, 16)])
                sums = jnp.where(lanes == i, jnp.sum(a) + sums, sums)
            outv[pl.ds(j * _CHUNK + g * 16, 16)] = sums

    pltpu.sync_copy(outv, out_hbm.at[pl.ds(base, _BPW)])


@jax.jit
def kernel(user_id, item_id, P, Q, user_bias, item_bias):
    mesh = plsc.VectorSubcoreMesh(core_axis_name="c", subcore_axis_name="s")

    detile = pl.kernel(
        _detile_body,
        out_type=(jax.ShapeDtypeStruct((_NPAIR, 128), jnp.float32),
                  jax.ShapeDtypeStruct((_NPAIR, 128), jnp.float32)),
        mesh=mesh,
        compiler_params=pltpu.CompilerParams(
            needs_layout_passes=False, use_tc_tiling_on_sc=True),
        scratch_types=[
            pltpu.VMEM((2, 64, 256), jnp.float32),
            pltpu.VMEM((2, 128, 128), jnp.float32),
            pltpu.VMEM((64, 64), jnp.float32),
            pltpu.VMEM((32, 128), jnp.float32),
            pltpu.SemaphoreType.DMA,
            pltpu.SemaphoreType.DMA,
            pltpu.SemaphoreType.DMA,
            pltpu.SemaphoreType.DMA,
        ],
    )
    cp, cq = detile(P.T, Q.T)

    bias = pl.kernel(
        _bias_body,
        out_type=jax.ShapeDtypeStruct((_BATCH,), jnp.float32),
        mesh=mesh,
        compiler_params=pltpu.CompilerParams(
            needs_layout_passes=False, use_tc_tiling_on_sc=False),
        scratch_types=[
            pltpu.VMEM((_NCHUNK, _CHUNK), jnp.int32),
            pltpu.VMEM((_NCHUNK, _CHUNK), jnp.int32),
            pltpu.VMEM((_NCHUNK, _CHUNK), jnp.int32),
            pltpu.VMEM((_NCHUNK, _CHUNK), jnp.int32),
            pltpu.VMEM((_BPW, 16), jnp.float32),
            pltpu.VMEM((_BPW, 16), jnp.float32),
            pltpu.VMEM((_BPW,), jnp.float32),
            pltpu.SemaphoreType.DMA,
        ],
    )
    bsum = bias(user_id, item_id,
                user_bias.reshape(-1, 16), item_bias.reshape(-1, 16))

    dots = pl.kernel(
        _dot_body,
        out_type=jax.ShapeDtypeStruct((_BATCH,), jnp.float32),
        mesh=mesh,
        compiler_params=pltpu.CompilerParams(
            needs_layout_passes=False, use_tc_tiling_on_sc=True),
        scratch_types=[
            pltpu.VMEM((_NCHUNK, _CHUNK), jnp.int32),
            pltpu.VMEM((_NCHUNK, _CHUNK), jnp.int32),
            pltpu.VMEM((_NCHUNK, _CHUNK), jnp.int32),
            pltpu.VMEM((_NCHUNK, _CHUNK), jnp.int32),
            pltpu.VMEM((2, _CHUNK, 128), jnp.float32),
            pltpu.VMEM((2, _CHUNK, 128), jnp.float32),
            pltpu.VMEM((_BPW,), jnp.float32),
            pltpu.VMEM((_BPW,), jnp.float32),
            pltpu.SemaphoreType.DMA,
            pltpu.SemaphoreType.DMA,
            pltpu.SemaphoreType.DMA,
            pltpu.SemaphoreType.DMA,
        ],
    )
    return dots(user_id, item_id, cp, cq, bsum)


# speedup vs baseline: 4.9115x; 1.0399x over previous
import jax
import jax.numpy as jnp
from jax import lax
from jax.experimental import pallas as pl
from jax.experimental.pallas import tpu as pltpu
from jax.experimental.pallas import tpu_sc as plsc

_BATCH = 16384
_F = 64
_NU = 1000000
_NC = 2
_NS = 16
_NW = _NC * _NS
_BPW = _BATCH // _NW
_CHUNK = 128
_NCHUNK = _BPW // _CHUNK
_NBLK = (_NU + 127) // 128
_LASTU0 = _NU - 128
_NPAIR = _NU // 2


def _wid():
    return lax.axis_index("s") * _NC + lax.axis_index("c")


def _detile_body(pt_hbm, qt_hbm, cp_hbm, cq_hbm,
                 insc, outsc, insc2, outsc2, s_in0, s_in1, s_out0, s_out1):
    wid = _wid()
    nfull = _NU // 256
    nblk = (nfull // _NW) + jnp.where(wid < nfull % _NW, 1, 0)

    lanes16 = lax.iota(jnp.int32, 16)
    uh = [(l0 * 16 + lanes16) >> 1 for l0 in range(8)]
    par6 = [((l0 * 16 + lanes16) & 1) << 6 for l0 in range(8)]
    fdiag = [(lanes16 + d) & 15 for d in range(16)]
    upar = (lanes16 & 1) << 6
    uhalf = lanes16 >> 1

    for (src, dst, s_in, s_out) in ((pt_hbm, cp_hbm, s_in0, s_out0),
                                    (qt_hbm, cq_hbm, s_in1, s_out1)):
        def u0_of(k):
            return pl.multiple_of((wid + k * _NW) * 256, 128)

        def start_in(k, b):
            pltpu.make_async_copy(
                src.at[:, pl.ds(u0_of(k), 256)], insc.at[b], s_in).start()

        def wait_in(b):
            pltpu.make_async_copy(
                src.at[:, pl.ds(0, 128)], insc.at[b], s_in).wait()

        def start_out(k, bo):
            r0 = pl.multiple_of(u0_of(k) >> 1, 8)
            pltpu.make_async_copy(
                outsc.at[bo], dst.at[pl.ds(r0, 128), :], s_out).start()

        def wait_out(bo):
            pltpu.make_async_copy(
                outsc.at[bo], dst.at[pl.ds(0, 128), :], s_out).wait()

        start_in(0, 0)

        @pl.when(nblk >= 2)
        def _():
            start_in(1, 1)

        def blk(k, carry):
            b = k % 3
            bo = k % 2

            @pl.when(k + 2 < nblk)
            def _():
                start_in(k + 2, (k + 2) % 3)

            wait_in(b)

            @pl.when(k >= 2)
            def _():
                wait_out(bo)

            @plsc.parallel_loop(0, 16, step=1, unroll=8)
            def frow(d):
                fd = (lanes16 + d) & 15
                for u0 in range(0, 256, 16):
                    u_vec = u0 + lanes16
                    r_vec = (u0 >> 1) + uhalf
                    for f0 in range(0, 64, 16):
                        f_vec = f0 + fd
                        v = plsc.load_gather(insc.at[b], [f_vec, u_vec])
                        plsc.store_scatter(outsc.at[bo], [r_vec, f_vec + upar], v)

            start_out(k, bo)
            return carry

        lax.fori_loop(0, nblk, blk, 0)

        for t in range(2):
            @pl.when(nblk >= 2 - t)
            def _():
                wait_out((nblk + t) % 2)

        @pl.when(wid == 0)
        def _():
            pltpu.sync_copy(src.at[:, pl.ds(_NU - 64, 64)], insc2)

            @plsc.parallel_loop(0, 64, step=1, unroll=8)
            def frow2(fr):
                for l0 in range(4):
                    v = insc2[fr, pl.ds(l0 * 16, 16)]
                    plsc.store_scatter(outsc2, [uh[l0], par6[l0] + fr], v)
            pltpu.sync_copy(outsc2, dst.at[pl.ds(_NPAIR - 32, 32), :])


def _bias_body(uid_hbm, iid_hbm, bu_hbm, bi_hbm, out_hbm,
               uidx, iidx, uhi, ihi, burows, birows, outv, sem):
    wid = _wid()
    base = wid * _BPW

    for j in range(_NCHUNK):
        pltpu.sync_copy(uid_hbm.at[pl.ds(base + j * _CHUNK, _CHUNK)], uidx.at[j])
        pltpu.sync_copy(iid_hbm.at[pl.ds(base + j * _CHUNK, _CHUNK)], iidx.at[j])

    for j in range(_NCHUNK):
        for t in range(_CHUNK // 16):
            sl = pl.ds(t * 16, 16)
            uhi.at[j][sl] = lax.shift_right_logical(uidx.at[j][sl], 4)
            ihi.at[j][sl] = lax.shift_right_logical(iidx.at[j][sl], 4)

    copies = []
    for j in range(_NCHUNK):
        sl = pl.ds(j * _CHUNK, _CHUNK)
        copies.append(pltpu.async_copy(bu_hbm.at[uhi.at[j]], burows.at[sl], sem))
        copies.append(pltpu.async_copy(bi_hbm.at[ihi.at[j]], birows.at[sl], sem))
    for cp in copies:
        cp.wait()

    lanes = lax.iota(jnp.int32, 16)

    def group(g, carry):
        rb = g * 16
        j = g // (_CHUNK // 16)
        o = (g % (_CHUNK // 16)) * 16
        rows = rb + lanes
        uvals = uidx.at[j][pl.ds(o, 16)]
        ivals = iidx.at[j][pl.ds(o, 16)]
        bu_v = plsc.load_gather(burows, [rows, jnp.bitwise_and(uvals, 15)])
        bi_v = plsc.load_gather(birows, [rows, jnp.bitwise_and(ivals, 15)])
        outv[pl.ds(rb, 16)] = bu_v + bi_v
        return carry

    lax.fori_loop(0, _BPW // 16, group, 0)
    pltpu.sync_copy(outv, out_hbm.at[pl.ds(base, _BPW)])


def _dot_body(uid_hbm, iid_hbm, cp_hbm, cq_hbm, bsum_hbm, out_hbm,
              uidx, iidx, upr, ipr, pbuf, qbuf, bsum, outv, s_p0, s_p1,
              s_q0, s_q1):
    wid = _wid()
    base = wid * _BPW

    for j in range(_NCHUNK):
        pltpu.sync_copy(uid_hbm.at[pl.ds(base + j * _CHUNK, _CHUNK)], uidx.at[j])
        pltpu.sync_copy(iid_hbm.at[pl.ds(base + j * _CHUNK, _CHUNK)], iidx.at[j])
    pltpu.sync_copy(bsum_hbm.at[pl.ds(base, _BPW)], bsum)

    for j in range(_NCHUNK):
        for t in range(_CHUNK // 16):
            sl = pl.ds(t * 16, 16)
            upr.at[j][sl] = lax.shift_right_logical(uidx.at[j][sl], 1)
            ipr.at[j][sl] = lax.shift_right_logical(iidx.at[j][sl], 1)

    sems = ((s_p0, s_q0), (s_p1, s_q1))

    def fire(j):
        b = j % 2
        pltpu.make_async_copy(cp_hbm.at[upr.at[j]], pbuf.at[b], sems[b][0]).start()
        pltpu.make_async_copy(cq_hbm.at[ipr.at[j]], qbuf.at[b], sems[b][1]).start()

    def drain(b):
        pltpu.make_async_copy(cp_hbm.at[upr.at[0]], pbuf.at[b], sems[b][0]).wait()
        pltpu.make_async_copy(cq_hbm.at[ipr.at[0]], qbuf.at[b], sems[b][1]).wait()

    lanes = lax.iota(jnp.int32, 16)
    fire(0)
    for j in range(_NCHUNK):
        if j + 1 < _NCHUNK:
            fire(j + 1)
        b = j % 2
        drain(b)
        for g in range(_CHUNK // 16):
            uvals = uidx.at[j][pl.ds(g * 16, 16)]
            ivals = iidx.at[j][pl.ds(g * 16, 16)]
            sums = bsum[pl.ds(j * _CHUNK + g * 16, 16)]
            for i in range(16):
                r = g * 16 + i
                hu = jnp.bitwise_and(uvals[i], 1) * 64
                hi_ = jnp.bitwise_and(ivals[i], 1) * 64
                a = (pbuf[b, r, pl.ds(hu, 16)] * qbuf[b, r, pl.ds(hi_, 16)])
                for k in range(1, _F // 16):
                    a = a + (pbuf[b, r, pl.ds(hu + 16 * k, 16)]
                             * qbuf[b, r, pl.ds(hi_ + 16 * k, 16)])
                sums = jnp.where(lanes == i, jnp.sum(a) + sums, sums)
            outv[pl.ds(j * _CHUNK + g * 16, 16)] = sums

    pltpu.sync_copy(outv, out_hbm.at[pl.ds(base, _BPW)])


@jax.jit
def kernel(user_id, item_id, P, Q, user_bias, item_bias):
    mesh = plsc.VectorSubcoreMesh(core_axis_name="c", subcore_axis_name="s")

    detile = pl.kernel(
        _detile_body,
        out_type=(jax.ShapeDtypeStruct((_NPAIR, 128), jnp.float32),
                  jax.ShapeDtypeStruct((_NPAIR, 128), jnp.float32)),
        mesh=mesh,
        compiler_params=pltpu.CompilerParams(
            needs_layout_passes=False, use_tc_tiling_on_sc=True),
        scratch_types=[
            pltpu.VMEM((3, 64, 256), jnp.float32),
            pltpu.VMEM((2, 128, 128), jnp.float32),
            pltpu.VMEM((64, 64), jnp.float32),
            pltpu.VMEM((32, 128), jnp.float32),
            pltpu.SemaphoreType.DMA,
            pltpu.SemaphoreType.DMA,
            pltpu.SemaphoreType.DMA,
            pltpu.SemaphoreType.DMA,
        ],
    )
    cp, cq = detile(P.T, Q.T)

    bias = pl.kernel(
        _bias_body,
        out_type=jax.ShapeDtypeStruct((_BATCH,), jnp.float32),
        mesh=mesh,
        compiler_params=pltpu.CompilerParams(
            needs_layout_passes=False, use_tc_tiling_on_sc=False),
        scratch_types=[
            pltpu.VMEM((_NCHUNK, _CHUNK), jnp.int32),
            pltpu.VMEM((_NCHUNK, _CHUNK), jnp.int32),
            pltpu.VMEM((_NCHUNK, _CHUNK), jnp.int32),
            pltpu.VMEM((_NCHUNK, _CHUNK), jnp.int32),
            pltpu.VMEM((_BPW, 16), jnp.float32),
            pltpu.VMEM((_BPW, 16), jnp.float32),
            pltpu.VMEM((_BPW,), jnp.float32),
            pltpu.SemaphoreType.DMA,
        ],
    )
    bsum = bias(user_id, item_id,
                user_bias.reshape(-1, 16), item_bias.reshape(-1, 16))

    dots = pl.kernel(
        _dot_body,
        out_type=jax.ShapeDtypeStruct((_BATCH,), jnp.float32),
        mesh=mesh,
        compiler_params=pltpu.CompilerParams(
            needs_layout_passes=False, use_tc_tiling_on_sc=True),
        scratch_types=[
            pltpu.VMEM((_NCHUNK, _CHUNK), jnp.int32),
            pltpu.VMEM((_NCHUNK, _CHUNK), jnp.int32),
            pltpu.VMEM((_NCHUNK, _CHUNK), jnp.int32),
            pltpu.VMEM((_NCHUNK, _CHUNK), jnp.int32),
            pltpu.VMEM((2, _CHUNK, 128), jnp.float32),
            pltpu.VMEM((2, _CHUNK, 128), jnp.float32),
            pltpu.VMEM((_BPW,), jnp.float32),
            pltpu.VMEM((_BPW,), jnp.float32),
            pltpu.SemaphoreType.DMA,
            pltpu.SemaphoreType.DMA,
            pltpu.SemaphoreType.DMA,
            pltpu.SemaphoreType.DMA,
        ],
    )
    return dots(user_id, item_id, cp, cq, bsum)
